# Initial kernel scaffold; baseline (speedup 1.0000x reference)
import jax
import jax.numpy as jnp
from jax.experimental import pallas as pl


def kernel(X, edge_index, edge_weight, concat_layers, y_initial, H, C, W_ih, W_hh, b_ih, b_hh, W_gnn1, b_gnn1, W_go1, b_go1, W_go2, b_go2, g_h, be_h, g_c, be_c):
    raise NotImplementedError("write your pallas kernel here")


# speedup vs baseline: 1.3956x; 1.3956x over previous
import functools

import jax
import jax.numpy as jnp
from jax import lax
from jax.experimental import pallas as pl
from jax.experimental.pallas import tpu as pltpu

N = 50000
E = 1600000
D_IN = 128
H_DIM = 256
OUT_DIM = 2

_R = 400


def _lstm_body(x_ref, h_ref, c_ref, wih_ref, whh_ref, b_ref,
               gh_ref, beh_ref, gc_ref, bec_ref,
               hln_ref, cln_ref, out_ref):
    x = x_ref[...]
    h = h_ref[...]
    c = c_ref[...]
    gates = (jnp.dot(x, wih_ref[...], preferred_element_type=jnp.float32)
             + jnp.dot(h, whh_ref[...], preferred_element_type=jnp.float32)
             + b_ref[...])
    i = jax.nn.sigmoid(gates[:, 0 * H_DIM:1 * H_DIM])
    f = jax.nn.sigmoid(gates[:, 1 * H_DIM:2 * H_DIM])
    g = jnp.tanh(gates[:, 2 * H_DIM:3 * H_DIM])
    o = jax.nn.sigmoid(gates[:, 3 * H_DIM:4 * H_DIM])
    c_new = f * c + i * g
    h_new = o * jnp.tanh(c_new)

    def _ln(v, gamma, beta):
        mu = jnp.mean(v, axis=-1, keepdims=True)
        var = jnp.mean((v - mu) ** 2, axis=-1, keepdims=True)
        return (v - mu) * lax.rsqrt(var + 1e-5) * gamma + beta

    hln_ref[...] = _ln(h_new, gh_ref[...], beh_ref[...])
    cln_ref[...] = _ln(c_new, gc_ref[...], bec_ref[...])
    out_ref[...] = jnp.where(h_new >= 0, h_new, 0.01 * h_new)


def _lstm_stage(X, H0, C0, wih_t, whh_t, bias, g_h, be_h, g_c, be_c):
    grid = (N // _R,)
    row = pl.BlockSpec((_R, None), lambda i: (i, 0))

    def full(shape):
        return pl.BlockSpec(shape, lambda i: tuple(0 for _ in shape))

    return pl.pallas_call(
        _lstm_body,
        grid=grid,
        in_specs=[
            pl.BlockSpec((_R, D_IN), lambda i: (i, 0)),
            pl.BlockSpec((_R, H_DIM), lambda i: (i, 0)),
            pl.BlockSpec((_R, H_DIM), lambda i: (i, 0)),
            full((D_IN, 4 * H_DIM)),
            full((H_DIM, 4 * H_DIM)),
            full((1, 4 * H_DIM)),
            full((1, H_DIM)), full((1, H_DIM)),
            full((1, H_DIM)), full((1, H_DIM)),
        ],
        out_specs=[
            pl.BlockSpec((_R, H_DIM), lambda i: (i, 0)),
            pl.BlockSpec((_R, H_DIM), lambda i: (i, 0)),
            pl.BlockSpec((_R, H_DIM), lambda i: (i, 0)),
        ],
        out_shape=[
            jax.ShapeDtypeStruct((N, H_DIM), jnp.float32),
            jax.ShapeDtypeStruct((N, H_DIM), jnp.float32),
            jax.ShapeDtypeStruct((N, H_DIM), jnp.float32),
        ],
    )(X, H0, C0, wih_t, whh_t, bias, g_h, be_h, g_c, be_c)


def kernel(X, edge_index, edge_weight, concat_layers, y_initial, H, C,
           W_ih, W_hh, b_ih, b_hh, W_gnn1, b_gnn1, W_go1, b_go1,
           W_go2, b_go2, g_h, be_h, g_c, be_c):
    src = edge_index[0].astype(jnp.int32)
    dst = edge_index[1].astype(jnp.int32)
    w = edge_weight

    deg = jnp.zeros((N,), jnp.float32).at[dst].add(w) + 1.0
    dinv = lax.rsqrt(deg)
    invdeg = 1.0 / deg
    norm = dinv[src] * w * dinv[dst]

    def aop(x):
        agg = jnp.zeros(x.shape, jnp.float32).at[dst].add(norm[:, None] * x[src])
        return agg + x * invdeg[:, None]

    bias = (b_ih + b_hh).reshape(1, -1)
    h_ln, c_ln, out_leaky = _lstm_stage(
        X, H[0], C[0], W_ih.T, W_hh.T, bias,
        g_h.reshape(1, -1), be_h.reshape(1, -1),
        g_c.reshape(1, -1), be_c.reshape(1, -1))

    s1 = aop(concat_layers)
    gnn1 = s1 @ W_gnn1.T + b_gnn1
    gnn1 = jnp.where(gnn1 >= 0, gnn1, 0.01 * gnn1)
    output = out_leaky + gnn1

    s2 = aop(output)
    out2 = s2 @ W_go1.T + b_go1
    out2 = jnp.where(out2 >= 0, out2, 0.01 * out2)

    m3 = out2 @ W_go2.T
    s3 = aop(m3)
    out3 = s3 + b_go2 + y_initial

    sic = jax.nn.sigmoid(out3[:, 0:1])
    out = jnp.concatenate([sic, out3[:, 1:2]], axis=-1)
    return (out, h_ln[None], c_ln[None])


# speedup vs baseline: 4.8216x; 3.4548x over previous
import functools

import jax
import jax.numpy as jnp
from jax import lax
from jax.experimental import pallas as pl
from jax.experimental.pallas import tpu as pltpu
from jax.experimental.pallas import tpu_sc as plsc

N = 50000
E = 1600000
D_IN = 128
H_DIM = 256

_R = 400
_NC = 2
_NS = 16
_NW = _NC * _NS
_L = 16
_EPT = 51200
_EPAD = _NW * _EPT
_MAC = 1024
_NMAC = _EPT // _MAC
_NSUB = _MAC // 128
_NPAD = 50048
_NPT = _NPAD // _NS

_IOTA = lambda: lax.iota(jnp.int32, _L)


def _wid():
    return lax.axis_index("c") * _NS + lax.axis_index("s")


def _mesh():
    return plsc.VectorSubcoreMesh(core_axis_name="c", subcore_axis_name="s")


_SC_PARAMS = pltpu.CompilerParams(needs_layout_passes=False,
                                  use_tc_tiling_on_sc=False)


@functools.partial(
    pl.kernel, mesh=_mesh(), compiler_params=_SC_PARAMS,
    out_type=jax.ShapeDtypeStruct((_NW * N,), jnp.float32),
    scratch_types=[
        pltpu.VMEM((N,), jnp.float32),
        pltpu.VMEM((_NSUB, 128), jnp.int32),
        pltpu.VMEM((_MAC,), jnp.float32),
    ])
def _sc_deg(dst2_hbm, w_hbm, out_hbm, acc, dstv2, wv):
    wid = _wid()

    def zero(i, _):
        acc[pl.ds(i * _L, _L)] = jnp.zeros((_L,), jnp.float32)
        return 0
    lax.fori_loop(0, N // _L, zero, 0)

    ebase = wid * _EPT

    def mac(m, _):
        e0 = ebase + m * _MAC
        r0 = pl.multiple_of(e0 // 128, _NSUB)
        pltpu.sync_copy(dst2_hbm.at[pl.ds(r0, _NSUB)], dstv2)
        pltpu.sync_copy(w_hbm.at[pl.ds(e0, _MAC)], wv)

        def grp(g, _):
            ii = _IOTA() + g * _L
            idx = plsc.load_gather(
                dstv2, [jnp.full((_L,), g // 8, jnp.int32),
                        _IOTA() + (g % 8) * _L])
            val = plsc.load_gather(wv, [ii])
            plsc.addupdate_scatter(acc, [idx], val)
            return 0
        lax.fori_loop(0, _MAC // _L, grp, 0)
        return 0
    lax.fori_loop(0, _NMAC, mac, 0)
    pltpu.sync_copy(acc, out_hbm.at[pl.ds(wid * N, N)])


def _conv_phase(table_hbm, src_hbm, dst2_hbm, norm_hbm, out_hbm,
                rows, srcv, dstv2, normv, zbuf, idxoff, f_width, n_chunks,
                mac):
    cid = lax.axis_index("c")
    sid = lax.axis_index("s")
    wid = cid * _NS + sid
    ebase = wid * _EPT
    nsub = mac // 128
    nmac = _EPT // mac
    nsub_z = _NPT // 184

    def zrow(i, _):
        ri = jnp.full((_L,), i, jnp.int32)
        for h in range(f_width // _L):
            plsc.store_scatter(zbuf, [ri, _IOTA() + h * _L],
                               jnp.zeros((_L,), jnp.float32))
        return 0
    lax.fori_loop(0, 184, zrow, 0)

    def chunk(c, acc_sh):
        def zcopy(j, _):
            r0 = pl.multiple_of(sid * _NPT + j * 184, 8)
            pltpu.sync_copy(zbuf, acc_sh.at[pl.ds(r0, 184)])
            return 0
        lax.fori_loop(0, nsub_z, zcopy, 0)
        plsc.subcore_barrier()

        def macb(m, _):
            e0 = ebase + m * mac
            pltpu.sync_copy(src_hbm.at[pl.ds(e0, mac)], srcv)
            r0 = pl.multiple_of(e0 // 128, nsub)
            pltpu.sync_copy(dst2_hbm.at[pl.ds(r0, nsub)], dstv2)
            pltpu.sync_copy(norm_hbm.at[pl.ds(e0, mac)], normv)
            if n_chunks > 1:
                def off(g, _):
                    ii = _IOTA() + g * _L
                    v = plsc.load_gather(srcv, [ii]) * n_chunks + c
                    plsc.store_scatter(idxoff, [ii], v)
                    return 0
                lax.fori_loop(0, mac // _L, off, 0)
                gsrc = idxoff
            else:
                gsrc = srcv

            def sub(j, _):
                pltpu.sync_copy(
                    table_hbm.at[gsrc.at[pl.ds(j * 128, 128)]],
                    rows.at[pl.ds(j * 128, 128)])
                return 0
            lax.fori_loop(0, nsub, sub, 0)

            def scale(i, _):
                ri = jnp.full((_L,), i, jnp.int32)
                nv = plsc.load_gather(normv, [ri])
                for h in range(f_width // _L):
                    col = _IOTA() + h * _L
                    v = plsc.load_gather(rows, [ri, col]) * nv
                    plsc.store_scatter(rows, [ri, col], v)
                return 0
            lax.fori_loop(0, mac, scale, 0)

            def scat(j, _):
                pltpu.sync_copy(
                    rows.at[pl.ds(j * 128, 128)],
                    acc_sh.at[dstv2.at[j]], add=True)
                return 0
            lax.fori_loop(0, nsub, scat, 0)
            return 0
        lax.fori_loop(0, nmac, macb, 0)
        plsc.subcore_barrier()
        r0 = pl.multiple_of(sid * _NPT, 8)
        pltpu.sync_copy(
            acc_sh.at[pl.ds(r0, _NPT)],
            out_hbm.at[c, cid, pl.ds(r0, _NPT)])
        return acc_sh

    def run(acc_sh):
        for c in range(n_chunks):
            chunk(c, acc_sh)

    return run


def _make_sc_conv(f_width, n_chunks, with_norm, mac=_MAC):
    out_types = [jax.ShapeDtypeStruct((n_chunks, _NC, _NPAD, f_width),
                                      jnp.float32)]
    if with_norm:
        out_types = [jax.ShapeDtypeStruct((_EPAD,), jnp.float32)] + out_types
    scratch = [
        pltpu.VMEM_SHARED((_NPAD, f_width), jnp.float32),
        pltpu.VMEM((mac, f_width), jnp.float32),
        pltpu.VMEM((mac,), jnp.int32),
        pltpu.VMEM((mac // 128, 128), jnp.int32),
        pltpu.VMEM((mac,), jnp.float32),
        pltpu.VMEM((184, f_width), jnp.float32),
        pltpu.VMEM((mac,), jnp.int32),
    ]
    if with_norm:
        scratch = scratch + [pltpu.VMEM((N,), jnp.float32)]

        @functools.partial(pl.kernel, mesh=_mesh(), out_type=out_types,
                           compiler_params=_SC_PARAMS,
                           scratch_types=scratch)
        def k(table_hbm, src_hbm, dst2_hbm, w_hbm, dinv_hbm,
              norm_hbm, out_hbm, acc_sh, rows, srcv, dstv2, normv, zbuf,
              idxoff, dinv_v):
            wid = _wid()
            ebase = wid * _EPT
            pltpu.sync_copy(dinv_hbm, dinv_v)

            def mac_n(m, _):
                e0 = ebase + m * _MAC
                pltpu.sync_copy(src_hbm.at[pl.ds(e0, _MAC)], srcv)
                r0 = pl.multiple_of(e0 // 128, _NSUB)
                pltpu.sync_copy(dst2_hbm.at[pl.ds(r0, _NSUB)], dstv2)
                pltpu.sync_copy(w_hbm.at[pl.ds(e0, _MAC)], normv)

                def grp(g, _):
                    ii = _IOTA() + g * _L
                    s = plsc.load_gather(srcv, [ii])
                    d = plsc.load_gather(
                        dstv2, [jnp.full((_L,), g // 8, jnp.int32),
                                _IOTA() + (g % 8) * _L])
                    w = plsc.load_gather(normv, [ii])
                    nv = (plsc.load_gather(dinv_v, [s]) * w *
                          plsc.load_gather(dinv_v, [d]))
                    plsc.store_scatter(normv, [ii], nv)
                    return 0
                lax.fori_loop(0, _MAC // _L, grp, 0)
                pltpu.sync_copy(normv, norm_hbm.at[pl.ds(e0, _MAC)])
                return 0
            lax.fori_loop(0, _NMAC, mac_n, 0)
            plsc.subcore_barrier()
            run = _conv_phase(table_hbm, src_hbm, dst2_hbm, norm_hbm,
                              out_hbm, rows, srcv, dstv2, normv, zbuf,
                              idxoff, f_width, n_chunks, mac)
            run(acc_sh)
        return k

    @functools.partial(pl.kernel, mesh=_mesh(), out_type=out_types,
                       compiler_params=_SC_PARAMS,
                       scratch_types=scratch)
    def k(table_hbm, src_hbm, dst2_hbm, norm_hbm, out_hbm,
          acc_sh, rows, srcv, dstv2, normv, zbuf, idxoff):
        run = _conv_phase(table_hbm, src_hbm, dst2_hbm, norm_hbm,
                          out_hbm, rows, srcv, dstv2, normv, zbuf,
                          idxoff, f_width, n_chunks, mac)
        run(acc_sh)
    return k


_sc_norm_conv1 = _make_sc_conv(16, 1, with_norm=True)
_sc_conv2 = _make_sc_conv(32, 8, with_norm=False, mac=512)
_sc_conv3 = _make_sc_conv(16, 1, with_norm=False)


def _deg_body(degp_ref, dinv_ref, invdeg_ref):
    deg = jnp.sum(degp_ref[...], axis=0, keepdims=True) + 1.0
    dinv_ref[...] = lax.rsqrt(deg)
    invdeg_ref[...] = 1.0 / deg


def _deg_stage(degp):
    return pl.pallas_call(
        _deg_body,
        grid=(1,),
        in_specs=[pl.BlockSpec((_NW, N), lambda i: (0, 0))],
        out_specs=[pl.BlockSpec((1, N), lambda i: (0, 0)),
                   pl.BlockSpec((1, N), lambda i: (0, 0))],
        out_shape=[jax.ShapeDtypeStruct((1, N), jnp.float32),
                   jax.ShapeDtypeStruct((1, N), jnp.float32)],
    )(degp)


def _lstm_body(x_ref, h_ref, c_ref, wih_ref, whh_ref, b_ref,
               gh_ref, beh_ref, gc_ref, bec_ref,
               hln_ref, cln_ref, out_ref):
    x = x_ref[...]
    h = h_ref[...]
    c = c_ref[...]
    gates = (jnp.dot(x, wih_ref[...], preferred_element_type=jnp.float32)
             + jnp.dot(h, whh_ref[...], preferred_element_type=jnp.float32)
             + b_ref[...])
    i = jax.nn.sigmoid(gates[:, 0 * H_DIM:1 * H_DIM])
    f = jax.nn.sigmoid(gates[:, 1 * H_DIM:2 * H_DIM])
    g = jnp.tanh(gates[:, 2 * H_DIM:3 * H_DIM])
    o = jax.nn.sigmoid(gates[:, 3 * H_DIM:4 * H_DIM])
    c_new = f * c + i * g
    h_new = o * jnp.tanh(c_new)

    def _ln(v, gamma, beta):
        mu = jnp.mean(v, axis=-1, keepdims=True)
        var = jnp.mean((v - mu) ** 2, axis=-1, keepdims=True)
        return (v - mu) * lax.rsqrt(var + 1e-5) * gamma + beta

    hln_ref[...] = _ln(h_new, gh_ref[...], beh_ref[...])
    cln_ref[...] = _ln(c_new, gc_ref[...], bec_ref[...])
    out_ref[...] = jnp.where(h_new >= 0, h_new, 0.01 * h_new)


def _lstm_stage(X, H0, C0, wih_t, whh_t, bias, g_h, be_h, g_c, be_c):
    def full(shape):
        return pl.BlockSpec(shape, lambda i: tuple(0 for _ in shape))

    return pl.pallas_call(
        _lstm_body,
        grid=(N // _R,),
        in_specs=[
            pl.BlockSpec((_R, D_IN), lambda i: (i, 0)),
            pl.BlockSpec((_R, H_DIM), lambda i: (i, 0)),
            pl.BlockSpec((_R, H_DIM), lambda i: (i, 0)),
            full((D_IN, 4 * H_DIM)),
            full((H_DIM, 4 * H_DIM)),
            full((1, 4 * H_DIM)),
            full((1, H_DIM)), full((1, H_DIM)),
            full((1, H_DIM)), full((1, H_DIM)),
        ],
        out_specs=[
            pl.BlockSpec((_R, H_DIM), lambda i: (i, 0)),
            pl.BlockSpec((_R, H_DIM), lambda i: (i, 0)),
            pl.BlockSpec((_R, H_DIM), lambda i: (i, 0)),
        ],
        out_shape=[
            jax.ShapeDtypeStruct((N, H_DIM), jnp.float32),
            jax.ShapeDtypeStruct((N, H_DIM), jnp.float32),
            jax.ShapeDtypeStruct((N, H_DIM), jnp.float32),
        ],
    )(X, H0, C0, wih_t, whh_t, bias, g_h, be_h, g_c, be_c)


def _mix_body(p1_ref, cat_ref, invd_ref, w1_ref, b1_ref, outl_ref, out_ref):
    s1 = (p1_ref[0, 0] + p1_ref[0, 1]) + cat_ref[...] * invd_ref[...]
    g = jnp.dot(s1, w1_ref[...], preferred_element_type=jnp.float32) \
        + b1_ref[...]
    g = jnp.where(g >= 0, g, 0.01 * g)
    out_ref[...] = outl_ref[...] + g


def _mix_stage(p1, concat16, invdeg_col, w1t16, b1, out_leaky):
    return pl.pallas_call(
        _mix_body,
        grid=(N // _R,),
        in_specs=[
            pl.BlockSpec((1, _NC, _R, 16), lambda i: (0, 0, i, 0)),
            pl.BlockSpec((_R, 16), lambda i: (i, 0)),
            pl.BlockSpec((_R, 1), lambda i: (i, 0)),
            pl.BlockSpec((16, H_DIM), lambda i: (0, 0)),
            pl.BlockSpec((1, H_DIM), lambda i: (0, 0)),
            pl.BlockSpec((_R, H_DIM), lambda i: (i, 0)),
        ],
        out_specs=[pl.BlockSpec((_R, H_DIM), lambda i: (i, 0))],
        out_shape=[jax.ShapeDtypeStruct((N, H_DIM), jnp.float32)],
    )(p1, concat16, invdeg_col, w1t16, b1, out_leaky)


def _out2_body(p2_ref, outp_ref, invd_ref, w1r_ref, w1_ref, b1_ref, w2_ref,
               m3_ref):
    acc = jnp.dot(outp_ref[...] * invd_ref[...], w1_ref[...],
                  preferred_element_type=jnp.float32)
    for c in range(H_DIM // 32):
        s2c = p2_ref[c, 0] + p2_ref[c, 1]
        acc = acc + jnp.dot(s2c, w1r_ref[c],
                            preferred_element_type=jnp.float32)
    o = acc + b1_ref[...]
    o = jnp.where(o >= 0, o, 0.01 * o)
    m3_ref[...] = jnp.dot(o, w2_ref[...], preferred_element_type=jnp.float32)


def _out2_stage(p2, output, invdeg_col, wgo1r, wgo1t, bgo1, wgo2t16):
    NCH = H_DIM // 32
    return pl.pallas_call(
        _out2_body,
        grid=(N // _R,),
        in_specs=[
            pl.BlockSpec((NCH, _NC, _R, 32), lambda i: (0, 0, i, 0)),
            pl.BlockSpec((_R, H_DIM), lambda i: (i, 0)),
            pl.BlockSpec((_R, 1), lambda i: (i, 0)),
            pl.BlockSpec((NCH, 32, H_DIM), lambda i: (0, 0, 0)),
            pl.BlockSpec((H_DIM, H_DIM), lambda i: (0, 0)),
            pl.BlockSpec((1, H_DIM), lambda i: (0, 0)),
            pl.BlockSpec((H_DIM, 16), lambda i: (0, 0)),
        ],
        out_specs=[pl.BlockSpec((_R, 16), lambda i: (i, 0))],
        out_shape=[jax.ShapeDtypeStruct((N, 16), jnp.float32)],
    )(p2, output, invdeg_col, wgo1r, wgo1t, bgo1, wgo2t16)


def _final_body(p3_ref, m3_ref, invd_ref, y_ref, b2_ref, out_ref):
    v = (p3_ref[0, 0] + p3_ref[0, 1]) + m3_ref[...] * invd_ref[...]
    o = v[:, 0:2] + b2_ref[...] + y_ref[...]
    out_ref[...] = jnp.concatenate(
        [jax.nn.sigmoid(o[:, 0:1]), o[:, 1:2]], axis=-1)


def _final_stage(p3, m3_16, invdeg_col, y_initial, bgo2):
    return pl.pallas_call(
        _final_body,
        grid=(N // _R,),
        in_specs=[
            pl.BlockSpec((1, _NC, _R, 16), lambda i: (0, 0, i, 0)),
            pl.BlockSpec((_R, 16), lambda i: (i, 0)),
            pl.BlockSpec((_R, 1), lambda i: (i, 0)),
            pl.BlockSpec((_R, 2), lambda i: (i, 0)),
            pl.BlockSpec((1, 2), lambda i: (0, 0)),
        ],
        out_specs=[pl.BlockSpec((_R, 2), lambda i: (i, 0))],
        out_shape=[jax.ShapeDtypeStruct((N, 2), jnp.float32)],
    )(p3, m3_16, invdeg_col, y_initial, bgo2)


def kernel(X, edge_index, edge_weight, concat_layers, y_initial, H, C,
           W_ih, W_hh, b_ih, b_hh, W_gnn1, b_gnn1, W_go1, b_go1,
           W_go2, b_go2, g_h, be_h, g_c, be_c):
    src = edge_index[0].astype(jnp.int32)
    dst = edge_index[1].astype(jnp.int32)
    w = edge_weight

    pad = _EPAD - E
    srcp = jnp.concatenate([src, jnp.zeros((pad,), jnp.int32)])
    dstp = jnp.concatenate([dst, jnp.zeros((pad,), jnp.int32)])
    wp = jnp.concatenate([w, jnp.zeros((pad,), jnp.float32)])
    dstp2 = dstp.reshape(_EPAD // 128, 128)

    concat16 = jnp.pad(concat_layers, ((0, 0), (0, 16 - 3)))
    w1t16 = jnp.pad(W_gnn1.T, ((0, 16 - 3), (0, 0)))
    wgo2t16 = jnp.pad(W_go2.T, ((0, 0), (0, 16 - 2)))

    degp = _sc_deg(dstp2, wp).reshape(_NW, N)
    dinv_row, invdeg_row = _deg_stage(degp)
    dinv = dinv_row.reshape(N)
    invdeg_col = invdeg_row.reshape(N, 1)

    bias = (b_ih + b_hh).reshape(1, -1)
    h_ln, c_ln, out_leaky = _lstm_stage(
        X, H[0], C[0], W_ih.T, W_hh.T, bias,
        g_h.reshape(1, -1), be_h.reshape(1, -1),
        g_c.reshape(1, -1), be_c.reshape(1, -1))

    normp, p1 = _sc_norm_conv1(concat16, srcp, dstp2, wp, dinv)

    [output] = _mix_stage(p1, concat16, invdeg_col, w1t16,
                          b_gnn1.reshape(1, -1), out_leaky)

    [p2] = _sc_conv2(output.reshape(N * 8, 32), srcp, dstp2, normp)

    wgo1t = W_go1.T
    [m3_16] = _out2_stage(p2, output, invdeg_col,
                          wgo1t.reshape(8, 32, H_DIM), wgo1t,
                          b_go1.reshape(1, -1), wgo2t16)

    [p3] = _sc_conv3(m3_16, srcp, dstp2, normp)

    [out] = _final_stage(p3, m3_16, invdeg_col, y_initial,
                         b_go2.reshape(1, -1))
    return (out, h_ln[None], c_ln[None])
